# baseline (speedup 1.0000x reference)
import functools

import jax
import jax.numpy as jnp
import numpy as np
from jax import lax
from jax.experimental import pallas as pl
from jax.experimental.pallas import tpu as pltpu
from jax.experimental.pallas import tpu_sc as plsc

_H = 384
_W = 384
_B = 4
_Z = 64
_BANDS = 8
_BH = _H // _BANDS
_CELL = 48
_TBL = 96
_ZP = 528
_NC = 2
_NS = 16
_L = 16


_KR = _H // _NS
_OUTW = 2 * 8 * _L


_PACKW = 464


def _sc_paint_pool(packed):
    mesh = plsc.VectorSubcoreMesh(
        core_axis_name="c", subcore_axis_name="s",
        num_cores=_NC, num_subcores=_NS)

    @functools.partial(
        pl.kernel,
        out_type=jax.ShapeDtypeStruct((_NC * _NS, _OUTW), jnp.float32),
        mesh=mesh,
        scratch_types=[
            pltpu.VMEM((_PACKW,), jnp.int32),
            pltpu.VMEM((_ZP,), jnp.int32),
            pltpu.VMEM((_KR * _W,), jnp.int32),
            pltpu.VMEM((_L * _L,), jnp.float32),
            pltpu.VMEM((_OUTW + _L,), jnp.float32),
        ],
        compiler_params=pltpu.CompilerParams(
            use_tc_tiling_on_sc=False, needs_layout_passes=False),
    )
    def body(packed_hbm, out_hbm, zv, zrecv, winv, tmpv, outv):
        c = lax.axis_index("c")
        s = lax.axis_index("s")
        wid = s * _NC + c

        lanes = lax.iota(jnp.int32, _L)
        sent = jnp.full((_L,), 7 * _Z, jnp.int32)
        zerov = jnp.zeros((_L,), jnp.float32)

        def batch_body(lb, carry):
            bb = c * 2 + lb
            pltpu.sync_copy(packed_hbm.at[bb], zv)

            with jax.named_scope("sc_prep"):
                for chunk in range(_Z // _L):
                    zi = chunk * _L + lanes
                    fb = zi * 4
                    sy = jnp.maximum(plsc.load_gather(zv, [fb]), 0)
                    sx = jnp.maximum(plsc.load_gather(zv, [fb + 1]), 0)
                    ey = jnp.minimum(plsc.load_gather(zv, [fb + 2]), _H)
                    ex = jnp.minimum(plsc.load_gather(zv, [fb + 3]), _W)
                    mk = zv[pl.ds(4 * _Z + chunk * _L, _L)]
                    ok = (ey > sy) & (ex > sx) & (mk > 0)
                    zi8 = zi * 8
                    zeroi = jnp.zeros((_L,), jnp.int32)
                    plsc.store_scatter(zrecv, [zi8], jnp.where(ok, sy, zeroi))
                    plsc.store_scatter(zrecv, [zi8 + 1], sx)
                    plsc.store_scatter(zrecv, [zi8 + 2],
                                       jnp.where(ok, ey, zeroi))
                    plsc.store_scatter(zrecv, [zi8 + 3], ex)

            with jax.named_scope("sc_init"):
                @plsc.parallel_loop(0, _KR, 1)
                def init_row(k):
                    for cc in range(_W // _L):
                        winv[pl.ds(k * _W + cc * _L, _L)] = sent

            with jax.named_scope("sc_paint"):
                def zbody(z, cr):
                    f = zrecv[pl.ds(z * 8, _L)]
                    sy = f[0]
                    sx = f[1]
                    ey = f[2]
                    ex = f[3]
                    klo = jnp.maximum((sy - s + 15) >> 4, 0)
                    khi = jnp.minimum((ey - s + 15) >> 4, _KR)

                    @pl.when((klo < khi) & (sx < ex))
                    def _():
                        sxs = jnp.full((_L,), sx, jnp.int32)
                        exs = jnp.full((_L,), ex, jnp.int32)
                        zs = jnp.full((_L,), 2 * z + 5 * _Z, jnp.int32)
                        cxlo = sx // _L
                        cxhi = (ex + _L - 1) // _L

                        @plsc.parallel_loop(klo, khi, 1)
                        def kbody(k):
                            kb = jnp.full((_L,), k * _W, jnp.int32)

                            @plsc.parallel_loop(cxlo, cxhi, 1, unroll=2)
                            def cbody(cx):
                                xs = cx * _L + lanes
                                m = (xs >= sxs) & (xs < exs)
                                plsc.store_scatter(winv, [kb + xs], zs, mask=m)
                    return cr
                lax.fori_loop(0, _Z, zbody, 0)

            with jax.named_scope("sc_pool"):
                for i in range(8):
                    @plsc.parallel_loop(0, 3, 1, carry=(zerov,) * 16)
                    def accs(d, accs_in):
                        base = (3 * i + d) * _W
                        a = list(accs_in)
                        for j in range(8):
                            for cc in range(_CELL // _L):
                                w = winv[pl.ds(base + j * _CELL + cc * _L, _L)]
                                a[j] = a[j] + plsc.bitcast(
                                    plsc.load_gather(zv, [w]), jnp.float32)
                                a[8 + j] = a[8 + j] + plsc.bitcast(
                                    plsc.load_gather(zv, [w + 1]), jnp.float32)
                        return tuple(a)
                    for t in range(16):
                        tmpv[pl.ds(t * _L, _L)] = accs[t]
                    colidx = lanes * _L
                    tot = zerov
                    for l in range(_L):
                        tot = tot + plsc.load_gather(tmpv, [colidx + l])
                    obase = lb * 64 + i * 8
                    plsc.store_compressed(outv.at[pl.ds(obase, _L)], tot,
                                          mask=lanes < 8)
                    plsc.store_compressed(outv.at[pl.ds(obase + 128, _L)], tot,
                                          mask=lanes >= 8)
            return carry
        lax.fori_loop(0, 2, batch_body, 0)

        pltpu.sync_copy(outv.at[pl.ds(0, _OUTW)], out_hbm.at[wid])

    return body(packed)


def _conv_body(sums_ref, sel_ref, w1_ref, b1_ref, w2_ref, b2_ref, s_ref,
               out_ref):
    s = jnp.dot(sel_ref[...], sums_ref[...],
                preferred_element_type=jnp.float32)
    x = jnp.log1p(jnp.maximum(s, 0.0) * (1.0 / (_CELL * _CELL)))
    w1 = w1_ref[...]
    b1 = b1_ref[...]
    b2 = b2_ref[...]
    for b in range(_B):
        c, lb = b // 2, b % 2
        xb = jnp.stack([x[c, lb * 64:lb * 64 + 64],
                        x[c, 128 + lb * 64:128 + lb * 64 + 64]])
        h = jnp.dot(w1, xb, preferred_element_type=jnp.float32) + b1
        h = h * jax.nn.sigmoid(h)
        acc = jnp.broadcast_to(b2, (32, 64))
        for k in range(9):
            g = jnp.dot(w2_ref[k], h, preferred_element_type=jnp.float32)
            acc = acc + jnp.dot(g, s_ref[k], preferred_element_type=jnp.float32)
        out_ref[b] = acc


def _shift_mats():
    s = np.zeros((9, 64, 64), np.float32)
    for dy in range(3):
        for dx in range(3):
            k = dy * 3 + dx
            for y in range(8):
                for x in range(8):
                    yy, xx = y + dy - 1, x + dx - 1
                    if 0 <= yy < 8 and 0 <= xx < 8:
                        s[k, yy * 8 + xx, y * 8 + x] = 1.0
    return s


_S = _shift_mats()
_SEL = np.zeros((2, 32), np.float32)
_SEL[0, 0::2] = 1.0
_SEL[1, 1::2] = 1.0


def kernel(hist_BZ2, mask_BZ, fr_BZ4, H, W, W1, b1, W2, b2):
    packed = jnp.concatenate([
        fr_BZ4.astype(jnp.int32).reshape(_B, _Z * 4),
        mask_BZ.astype(jnp.int32),
        lax.bitcast_convert_type(hist_BZ2, jnp.int32).reshape(_B, _Z * 2),
        jnp.zeros((_B, _PACKW - 7 * _Z), jnp.int32),
    ], axis=1)
    sums = _sc_paint_pool(packed)
    w2m = jnp.transpose(W2, (2, 3, 0, 1)).reshape(9, 32, 64)
    out = pl.pallas_call(
        _conv_body,
        out_shape=jax.ShapeDtypeStruct((_B, 32, 64), jnp.float32),
    )(sums, jnp.asarray(_SEL), W1.reshape(64, 2), b1.reshape(64, 1), w2m,
      b2.reshape(32, 1), jnp.asarray(_S))
    del H, W
    return out.reshape(_B, 32, 8, 8)


# speedup vs baseline: 1.0531x; 1.0531x over previous
import functools

import jax
import jax.numpy as jnp
import numpy as np
from jax import lax
from jax.experimental import pallas as pl
from jax.experimental.pallas import tpu as pltpu
from jax.experimental.pallas import tpu_sc as plsc

_H = 384
_W = 384
_B = 4
_Z = 64
_BANDS = 8
_BH = _H // _BANDS
_CELL = 48
_TBL = 96
_ZP = 528
_NC = 2
_NS = 16
_L = 16


_KR = _H // _NS
_OUTW = 2 * 8 * _L


_PACKW = 464


def _sc_paint_pool(packed):
    mesh = plsc.VectorSubcoreMesh(
        core_axis_name="c", subcore_axis_name="s",
        num_cores=_NC, num_subcores=_NS)

    @functools.partial(
        pl.kernel,
        out_type=jax.ShapeDtypeStruct((_NC * _NS, _OUTW), jnp.float32),
        mesh=mesh,
        scratch_types=[
            pltpu.VMEM((_PACKW,), jnp.int32),
            pltpu.VMEM((_ZP,), jnp.int32),
            pltpu.VMEM((_KR * _W,), jnp.int32),
            pltpu.VMEM((_L * _L,), jnp.float32),
            pltpu.VMEM((_OUTW + _L,), jnp.float32),
        ],
        compiler_params=pltpu.CompilerParams(
            use_tc_tiling_on_sc=False, needs_layout_passes=False),
    )
    def body(packed_hbm, out_hbm, zv, zrecv, winv, tmpv, outv):
        c = lax.axis_index("c")
        s = lax.axis_index("s")
        wid = s * _NC + c

        lanes = lax.iota(jnp.int32, _L)
        sent = jnp.full((_L,), 7 * _Z, jnp.int32)
        zerov = jnp.zeros((_L,), jnp.float32)

        def batch_body(lb, carry):
            bb = c * 2 + lb
            pltpu.sync_copy(packed_hbm.at[bb], zv)

            with jax.named_scope("sc_prep"):
                cbase = jnp.zeros((_L,), jnp.int32)
                for chunk in range(_Z // _L):
                    zi = chunk * _L + lanes
                    fb = zi * 4
                    sy = jnp.maximum(plsc.load_gather(zv, [fb]), 0)
                    sx = jnp.maximum(plsc.load_gather(zv, [fb + 1]), 0)
                    ey = jnp.minimum(plsc.load_gather(zv, [fb + 2]), _H)
                    ex = jnp.minimum(plsc.load_gather(zv, [fb + 3]), _W)
                    mk = zv[pl.ds(4 * _Z + chunk * _L, _L)]
                    ok = (ey > sy) & (ex > sx) & (mk > 0)
                    p8 = (cbase + plsc.cumsum(ok.astype(jnp.int32)) - 1) * 8
                    plsc.store_scatter(zrecv, [p8], sy, mask=ok)
                    plsc.store_scatter(zrecv, [p8 + 1], sx, mask=ok)
                    plsc.store_scatter(zrecv, [p8 + 2], ey, mask=ok)
                    plsc.store_scatter(zrecv, [p8 + 3], ex, mask=ok)
                    plsc.store_scatter(zrecv, [p8 + 4], zi + zi + 5 * _Z,
                                       mask=ok)
                    cbase = cbase + plsc.all_reduce_population_count(ok)
                nvalid = cbase[0]

            with jax.named_scope("sc_init"):
                @plsc.parallel_loop(0, _KR, 1)
                def init_row(k):
                    for cc in range(_W // _L):
                        winv[pl.ds(k * _W + cc * _L, _L)] = sent

            with jax.named_scope("sc_paint"):
                def zbody(z, cr):
                    f = zrecv[pl.ds(z * 8, _L)]
                    sy = f[0]
                    sx = f[1]
                    ey = f[2]
                    ex = f[3]
                    widx = f[4]
                    klo = jnp.maximum((sy - s + 15) >> 4, 0)
                    khi = jnp.minimum((ey - s + 15) >> 4, _KR)

                    @pl.when(klo < khi)
                    def _():
                        sxs = jnp.full((_L,), sx, jnp.int32)
                        exs = jnp.full((_L,), ex, jnp.int32)
                        zs = jnp.full((_L,), widx, jnp.int32)
                        cxlo = sx // _L
                        cxhi = (ex + _L - 1) // _L

                        @plsc.parallel_loop(klo, khi, 1)
                        def kbody(k):
                            kb = jnp.full((_L,), k * _W, jnp.int32)

                            @plsc.parallel_loop(cxlo, cxhi, 1, unroll=2)
                            def cbody(cx):
                                xs = cx * _L + lanes
                                m = (xs >= sxs) & (xs < exs)
                                plsc.store_scatter(winv, [kb + xs], zs, mask=m)
                    return cr
                lax.fori_loop(0, nvalid, zbody, 0)

            with jax.named_scope("sc_pool"):
                for i in range(8):
                    @plsc.parallel_loop(0, 3, 1, carry=(zerov,) * 16)
                    def accs(d, accs_in):
                        base = (3 * i + d) * _W
                        a = list(accs_in)
                        for j in range(8):
                            for cc in range(_CELL // _L):
                                w = winv[pl.ds(base + j * _CELL + cc * _L, _L)]
                                a[j] = a[j] + plsc.bitcast(
                                    plsc.load_gather(zv, [w]), jnp.float32)
                                a[8 + j] = a[8 + j] + plsc.bitcast(
                                    plsc.load_gather(zv, [w + 1]), jnp.float32)
                        return tuple(a)
                    for t in range(16):
                        tmpv[pl.ds(t * _L, _L)] = accs[t]
                    colidx = lanes * _L
                    tot = zerov
                    for l in range(_L):
                        tot = tot + plsc.load_gather(tmpv, [colidx + l])
                    obase = lb * 64 + i * 8
                    plsc.store_compressed(outv.at[pl.ds(obase, _L)], tot,
                                          mask=lanes < 8)
                    plsc.store_compressed(outv.at[pl.ds(obase + 128, _L)], tot,
                                          mask=lanes >= 8)
            return carry
        lax.fori_loop(0, 2, batch_body, 0)

        pltpu.sync_copy(outv.at[pl.ds(0, _OUTW)], out_hbm.at[wid])

    return body(packed)


def _conv_body(sums_ref, sel_ref, w1_ref, b1_ref, w2_ref, b2_ref, s_ref,
               out_ref):
    s = jnp.dot(sel_ref[...], sums_ref[...],
                preferred_element_type=jnp.float32)
    x = jnp.log1p(jnp.maximum(s, 0.0) * (1.0 / (_CELL * _CELL)))
    w1 = w1_ref[...]
    b1 = b1_ref[...]
    b2 = b2_ref[...]
    for b in range(_B):
        c, lb = b // 2, b % 2
        xb = jnp.stack([x[c, lb * 64:lb * 64 + 64],
                        x[c, 128 + lb * 64:128 + lb * 64 + 64]])
        h = jnp.dot(w1, xb, preferred_element_type=jnp.float32) + b1
        h = h * jax.nn.sigmoid(h)
        acc = jnp.broadcast_to(b2, (32, 64))
        for k in range(9):
            g = jnp.dot(w2_ref[k], h, preferred_element_type=jnp.float32)
            acc = acc + jnp.dot(g, s_ref[k], preferred_element_type=jnp.float32)
        out_ref[b] = acc


def _shift_mats():
    s = np.zeros((9, 64, 64), np.float32)
    for dy in range(3):
        for dx in range(3):
            k = dy * 3 + dx
            for y in range(8):
                for x in range(8):
                    yy, xx = y + dy - 1, x + dx - 1
                    if 0 <= yy < 8 and 0 <= xx < 8:
                        s[k, yy * 8 + xx, y * 8 + x] = 1.0
    return s


_S = _shift_mats()
_SEL = np.zeros((2, 32), np.float32)
_SEL[0, 0::2] = 1.0
_SEL[1, 1::2] = 1.0


def kernel(hist_BZ2, mask_BZ, fr_BZ4, H, W, W1, b1, W2, b2):
    packed = jnp.concatenate([
        fr_BZ4.astype(jnp.int32).reshape(_B, _Z * 4),
        mask_BZ.astype(jnp.int32),
        lax.bitcast_convert_type(hist_BZ2, jnp.int32).reshape(_B, _Z * 2),
        jnp.zeros((_B, _PACKW - 7 * _Z), jnp.int32),
    ], axis=1)
    sums = _sc_paint_pool(packed)
    w2m = jnp.transpose(W2, (2, 3, 0, 1)).reshape(9, 32, 64)
    out = pl.pallas_call(
        _conv_body,
        out_shape=jax.ShapeDtypeStruct((_B, 32, 64), jnp.float32),
    )(sums, jnp.asarray(_SEL), W1.reshape(64, 2), b1.reshape(64, 1), w2m,
      b2.reshape(32, 1), jnp.asarray(_S))
    del H, W
    return out.reshape(_B, 32, 8, 8)
